# Initial kernel scaffold; baseline (speedup 1.0000x reference)
import jax
import jax.numpy as jnp
from jax.experimental import pallas as pl


def kernel(time_ids, pe_weight):
    raise NotImplementedError("write your pallas kernel here")


# speedup vs baseline: 3.5391x; 3.5391x over previous
import functools

import jax
import jax.numpy as jnp
from jax import lax
from jax.experimental import pallas as pl
from jax.experimental.pallas import tpu as pltpu
from jax.experimental.pallas import tpu_sc as plsc

D_MODEL = 64
NUM_WORKERS = 32
CHUNK = 128


def _gather_body(idx_hbm, table_hbm, out_hbm, idx_v, rows_v, gsem):
    n_chunks = idx_hbm.shape[1]
    wid = lax.axis_index("s") * 2 + lax.axis_index("c")
    pltpu.sync_copy(idx_hbm.at[wid], idx_v)
    base = wid * n_chunks * CHUNK

    def body(j, carry):
        pltpu.async_copy(table_hbm.at[idx_v.at[j]], rows_v, gsem).wait()
        pltpu.sync_copy(rows_v, out_hbm.at[pl.ds(base + j * CHUNK, CHUNK)])
        return carry

    lax.fori_loop(0, n_chunks, body, 0)


def kernel(time_ids, pe_weight):
    b, s = time_ids.shape
    total = b * s
    rows_per_w = total // NUM_WORKERS
    n_chunks = rows_per_w // CHUNK
    idx = time_ids.reshape(NUM_WORKERS, n_chunks, CHUNK)

    mesh = plsc.VectorSubcoreMesh(core_axis_name="c", subcore_axis_name="s")
    run = functools.partial(
        pl.kernel,
        mesh=mesh,
        out_type=jax.ShapeDtypeStruct((total, D_MODEL), jnp.float32),
        scratch_types=[
            pltpu.VMEM((n_chunks, CHUNK), jnp.int32),
            pltpu.VMEM((CHUNK, D_MODEL), jnp.float32),
            pltpu.SemaphoreType.DMA,
        ],
        compiler_params=pltpu.CompilerParams(use_tc_tiling_on_sc=False),
    )(_gather_body)
    out = run(idx, pe_weight)
    return out.reshape(b, s, D_MODEL)


# speedup vs baseline: 4.2668x; 1.2056x over previous
import functools

import jax
import jax.numpy as jnp
from jax import lax
from jax.experimental import pallas as pl
from jax.experimental.pallas import tpu as pltpu
from jax.experimental.pallas import tpu_sc as plsc

D_MODEL = 64
NUM_WORKERS = 32
CHUNK = 128
NBUF = 8


def _gather_body(idx_hbm, table_hbm, out_hbm, idx_v, rows, gsems, ssems):
    n_chunks = idx_hbm.shape[1]
    n_rounds = n_chunks // NBUF
    wid = lax.axis_index("s") * 2 + lax.axis_index("c")
    pltpu.sync_copy(idx_hbm.at[wid], idx_v)
    base = wid * n_chunks * CHUNK

    for b in range(NBUF):
        pltpu.async_copy(table_hbm.at[idx_v.at[b]], rows[b], gsems[b])

    def body(i, carry):
        for b in range(NBUF):
            j = i * NBUF + b
            pltpu.make_async_copy(table_hbm.at[idx_v.at[0]], rows[b],
                                  gsems[b]).wait()
            out_slice = out_hbm.at[pl.ds(base + j * CHUNK, CHUNK)]
            pltpu.async_copy(rows[b], out_slice, ssems[b])

            @pl.when(i < n_rounds - 1)
            def _():
                pltpu.make_async_copy(rows[b], out_slice, ssems[b]).wait()
                pltpu.async_copy(table_hbm.at[idx_v.at[j + NBUF]], rows[b],
                                 gsems[b])
        return carry

    lax.fori_loop(0, n_rounds, body, 0)
    for b in range(NBUF):
        last = (n_rounds - 1) * NBUF + b
        pltpu.make_async_copy(
            rows[b], out_hbm.at[pl.ds(base + last * CHUNK, CHUNK)],
            ssems[b]).wait()


def kernel(time_ids, pe_weight):
    b, s = time_ids.shape
    total = b * s
    rows_per_w = total // NUM_WORKERS
    n_chunks = rows_per_w // CHUNK
    idx = time_ids.reshape(NUM_WORKERS, n_chunks, CHUNK)

    mesh = plsc.VectorSubcoreMesh(core_axis_name="c", subcore_axis_name="s")
    run = functools.partial(
        pl.kernel,
        mesh=mesh,
        out_type=jax.ShapeDtypeStruct((total, D_MODEL), jnp.float32),
        scratch_types=[
            pltpu.VMEM((n_chunks, CHUNK), jnp.int32),
            [pltpu.VMEM((CHUNK, D_MODEL), jnp.float32) for _ in range(NBUF)],
            [pltpu.SemaphoreType.DMA for _ in range(NBUF)],
            [pltpu.SemaphoreType.DMA for _ in range(NBUF)],
        ],
        compiler_params=pltpu.CompilerParams(use_tc_tiling_on_sc=False),
    )(_gather_body)
    out = run(idx, pe_weight)
    return out.reshape(b, s, D_MODEL)
